# baseline (device time: 28805 ns/iter reference)
import jax
import jax.numpy as jnp
from jax import lax
from jax.experimental import pallas as pl
from jax.experimental.pallas import tpu as pltpu


def kernel(x, assign, W1, W2):
    t, d = x.shape
    n_exp, _, f = W1.shape
    assign2d = assign.reshape(t, 1)

    def body(x_ref, a_ref, w1_ref, w2_ref, out_ref,
             xs_ref, xr_ref, as_ref, ar_ref, ys_ref, yr_ref,
             send_sems, recv_sems):
        my_x = lax.axis_index("x")
        my_y = lax.axis_index("y")
        my_z = lax.axis_index("z")
        peer = (my_x, my_y, 1 - my_z)

        barrier = pltpu.get_barrier_semaphore()
        pl.semaphore_signal(barrier, inc=1, device_id=peer,
                            device_id_type=pl.DeviceIdType.MESH)
        pl.semaphore_wait(barrier, 1)

        xs_ref[...] = x_ref[...].astype(jnp.bfloat16)
        as_ref[...] = a_ref[...]

        rdma_x = pltpu.make_async_remote_copy(
            src_ref=xs_ref, dst_ref=xr_ref,
            send_sem=send_sems.at[0], recv_sem=recv_sems.at[0],
            device_id=peer, device_id_type=pl.DeviceIdType.MESH)
        rdma_x.start()
        rdma_a = pltpu.make_async_remote_copy(
            src_ref=as_ref, dst_ref=ar_ref,
            send_sem=send_sems.at[1], recv_sem=recv_sems.at[1],
            device_id=peer, device_id_type=pl.DeviceIdType.MESH)
        rdma_a.start()

        e_base = 2 * my_z

        def ffn(x_blk, a_blk):
            acc = jnp.zeros((t, d), jnp.float32)
            for el in range(n_exp):
                mask = a_blk == (e_base + el)
                xm = jnp.where(mask, x_blk, jnp.bfloat16(0))
                h = jnp.maximum(
                    jnp.dot(xm, w1_ref[el].astype(jnp.bfloat16),
                            preferred_element_type=jnp.float32),
                    0.0,
                )
                acc = acc + jnp.dot(
                    h.astype(jnp.bfloat16), w2_ref[el].astype(jnp.bfloat16),
                    preferred_element_type=jnp.float32)
            return acc

        out_ref[...] = ffn(xs_ref[...], a_ref[...])

        rdma_x.wait()
        rdma_a.wait()
        ys_ref[...] = ffn(xr_ref[...], ar_ref[...]).astype(jnp.bfloat16)

        rdma_y = pltpu.make_async_remote_copy(
            src_ref=ys_ref, dst_ref=yr_ref,
            send_sem=send_sems.at[2], recv_sem=recv_sems.at[2],
            device_id=peer, device_id_type=pl.DeviceIdType.MESH)
        rdma_y.start()
        rdma_y.wait()

        out_ref[...] = out_ref[...] + yr_ref[...].astype(jnp.float32)

    return pl.pallas_call(
        body,
        out_shape=jax.ShapeDtypeStruct((t, d), jnp.float32),
        in_specs=[
            pl.BlockSpec(memory_space=pltpu.VMEM),
            pl.BlockSpec(memory_space=pltpu.VMEM),
            pl.BlockSpec(memory_space=pltpu.VMEM),
            pl.BlockSpec(memory_space=pltpu.VMEM),
        ],
        out_specs=pl.BlockSpec(memory_space=pltpu.VMEM),
        scratch_shapes=[
            pltpu.VMEM((t, d), jnp.bfloat16),
            pltpu.VMEM((t, d), jnp.bfloat16),
            pltpu.VMEM((t, 1), jnp.int32),
            pltpu.VMEM((t, 1), jnp.int32),
            pltpu.VMEM((t, d), jnp.bfloat16),
            pltpu.VMEM((t, d), jnp.bfloat16),
            pltpu.SemaphoreType.DMA((3,)),
            pltpu.SemaphoreType.DMA((3,)),
        ],
        compiler_params=pltpu.CompilerParams(collective_id=0),
    )(x, assign2d, W1, W2)
